# baseline (speedup 1.0000x reference)
import jax
import jax.numpy as jnp
from jax.experimental import pallas as pl

N_NODES = 10000
N_GRAPHS = 128


def _tail(logits_ref, out_ref):
    l = logits_ref[...]
    m = jnp.max(l, axis=1, keepdims=True)
    e = jnp.exp(l - m)
    out_ref[...] = (l - m) - jnp.log(jnp.sum(e, axis=1, keepdims=True))


def kernel(x, edge_index, edge_weight, batch, coefs, conv_w, conv_b, fc_w, fc_b):
    src = edge_index[0]
    dst = edge_index[1]
    deg = jax.ops.segment_sum(edge_weight, dst, num_segments=N_NODES)
    safe = jnp.where(deg > 0, deg, 1.0)
    dis = jnp.where(deg > 0, jax.lax.rsqrt(safe), 0.0)
    norm = dis[src] * edge_weight * dis[dst]

    def Lmv(h):
        return -jax.ops.segment_sum(norm[:, None] * h[src], dst, num_segments=N_NODES)

    T = coefs.shape[0]
    DEG = coefs.shape[1] - 1
    Tx0 = x
    out = coefs[:, 0][:, None, None] * Tx0[None, :, :]
    Tx1 = Lmv(x)
    out = out + coefs[:, 1][:, None, None] * Tx1[None, :, :]
    for k in range(2, DEG + 1):
        Tx2 = 2.0 * Lmv(Tx1) - Tx0
        out = out + coefs[:, k][:, None, None] * Tx2[None, :, :]
        Tx0, Tx1 = Tx1, Tx2

    h = jnp.transpose(out, (1, 2, 0))
    h = jax.lax.conv_general_dilated(h, conv_w, (1,), 'VALID',
                                     dimension_numbers=('NCH', 'OIH', 'NCH'))
    h = h + conv_b[None, :, None]
    h = jax.lax.reduce_window(h, -jnp.inf, jax.lax.max, (1, 1, 2), (1, 1, 2), 'VALID')
    h = jnp.transpose(h, (2, 0, 1))
    h = jax.nn.relu(h)
    ht = jnp.transpose(h, (1, 0, 2))
    ssum = jax.ops.segment_sum(ht, batch, num_segments=N_GRAPHS)
    cnt = jax.ops.segment_sum(jnp.ones((N_NODES,), jnp.float32), batch, num_segments=N_GRAPHS)
    cnt = jnp.maximum(cnt, 1.0)
    gm = ssum / cnt[:, None, None]
    flat = gm.reshape(N_GRAPHS, -1)
    flat = jax.nn.relu(flat)
    logits = flat @ fc_w.T + fc_b
    return pl.pallas_call(
        _tail,
        out_shape=jax.ShapeDtypeStruct(logits.shape, logits.dtype),
    )(logits)


# speedup vs baseline: 1.0692x; 1.0692x over previous
import jax
import jax.numpy as jnp
from jax import lax
from jax.experimental import pallas as pl
from jax.experimental.pallas import tpu as pltpu
from jax.experimental.pallas import tpu_sc as plsc

NN = 10000
NNP = 10240
EE = 320000
QF = 32
EPT = 20000
B = 128
NB = EPT // B + (1 if EPT % B else 0)
EPTP = NB * B
NPT = NNP // 16
CH = 128
KDEG = 16
NROWS = 68 * NNP
TT = 16
NGR = 128
NBLK = 20
BN = NNP // NBLK


def _sc_body(x4f, srcp, dstp, wp, y_hbm,
             idx_t, dst_t, nrm_t, dis_v, rows0, rows1, zb2, zb1, wbuf,
             y_s, deg_s, dis_s, sem0, sem1):
    c = lax.axis_index("c")
    t = lax.axis_index("s")
    qbase = 2 * c

    pltpu.sync_copy(srcp.at[t], idx_t)
    pltpu.sync_copy(dstp.at[t], dst_t)

    zeros16 = jnp.zeros((16,), jnp.float32)

    def _zb2_body(r, _):
        for j in range(QF // 16):
            zb2[r, pl.ds(j * 16, 16)] = zeros16
        return 0
    lax.fori_loop(0, B, _zb2_body, 0, unroll=4)

    def _zb1_body(g, _):
        zb1[pl.ds(g * 16, 16)] = zeros16
        return 0
    lax.fori_loop(0, NPT // 16, _zb1_body, 0, unroll=4)

    pltpu.sync_copy(zb1, deg_s.at[pl.ds(t * NPT, NPT)])
    for ch in range(5):
        node0 = t * NPT + ch * CH
        pltpu.sync_copy(zb2, y_s.at[pl.ds(node0, CH)])
        for q in range(2):
            r0 = (qbase + q) * NNP + node0
            pltpu.sync_copy(x4f.at[pl.ds(r0, CH)], rows0)
            pltpu.sync_copy(rows0, y_hbm.at[pl.ds(r0, CH)])
    plsc.subcore_barrier()

    def _deg_body(b, _):
        pltpu.sync_copy(wp.at[pl.ds(t * EPTP + b * B, B)], wbuf)
        pltpu.sync_copy(wbuf, deg_s.at[dst_t.at[b]], add=True)
        return 0
    lax.fori_loop(0, NB, _deg_body, 0)
    plsc.subcore_barrier()

    pltpu.sync_copy(deg_s.at[pl.ds(t * NPT, NPT)], dis_v.at[pl.ds(0, NPT)])

    def _dis_body(g, _):
        v = dis_v[pl.ds(g * 16, 16)]
        i = lax.bitcast_convert_type(v, jnp.int32)
        i = 0x5F3759DF - lax.shift_right_arithmetic(i, 1)
        y = lax.bitcast_convert_type(i, jnp.float32)
        for _ in range(3):
            y = y * (1.5 - 0.5 * v * y * y)
        dis_v[pl.ds(g * 16, 16)] = jnp.where(v > 0.0, y, 0.0)
        return 0
    lax.fori_loop(0, NPT // 16, _dis_body, 0, unroll=4)
    pltpu.sync_copy(dis_v.at[pl.ds(0, NPT)], dis_s.at[pl.ds(t * NPT, NPT)])
    plsc.subcore_barrier()
    pltpu.sync_copy(dis_s, dis_v)

    def _nrm_body(b, _):
        pltpu.sync_copy(wp.at[pl.ds(t * EPTP + b * B, B)], wbuf)
        for g in range(8):
            sl = pl.ds(g * 16, 16)
            srcv = idx_t[b, sl]
            dstv = dst_t[b, sl]
            d1 = plsc.load_gather(dis_v, [srcv])
            d2 = plsc.load_gather(dis_v, [dstv])
            nrm_t[b, sl] = d1 * wbuf[sl] * d2
            idx_t[b, sl] = srcv + qbase * NNP
        return 0
    lax.fori_loop(0, NB, _nrm_body, 0)
    plsc.subcore_barrier()

    iota16 = lax.iota(jnp.int32, 16)

    def _scale_scatter(buf, b):
        def _grp(g, _):
            rowi = g * 16 + iota16
            nv = nrm_t[b, pl.ds(g * 16, 16)]

            def _f(f, _):
                colv = jnp.zeros((16,), jnp.int32) + f
                v = plsc.load_gather(buf, [rowi, colv])
                plsc.store_scatter(buf, [rowi, colv], v * nv)
                return 0
            lax.fori_loop(0, QF, _f, 0, unroll=8)
            return 0
        lax.fori_loop(0, B // 16, _grp, 0)
        pltpu.sync_copy(buf, y_s.at[dst_t.at[b]], add=True)

    def _shift_idx(delta):
        def _shift(b, _):
            for g in range(8):
                sl = pl.ds(g * 16, 16)
                idx_t[b, sl] = idx_t[b, sl] + delta
            return 0
        lax.fori_loop(0, NB, _shift, 0)

    def _kbody(k, _):
        alpha = jnp.where(k == 1, -1.0, -2.0)
        gamma = jnp.where(k == 1, 0.0, -1.0)
        km2 = jnp.where(k == 1, 0, k - 2)
        for q in range(2):
            qq = qbase + q
            pltpu.async_copy(y_hbm.at[idx_t.at[0]], rows0, sem0)

            def _pair(p, _):
                b0 = 2 * p
                b1 = b0 + 1
                pltpu.async_copy(y_hbm.at[idx_t.at[b1]], rows1, sem1)
                pltpu.make_async_copy(y_hbm.at[idx_t.at[b0]], rows0, sem0).wait()
                _scale_scatter(rows0, b0)
                pltpu.async_copy(y_hbm.at[idx_t.at[b0 + 2]], rows0, sem0)
                pltpu.make_async_copy(y_hbm.at[idx_t.at[b1]], rows1, sem1).wait()
                _scale_scatter(rows1, b1)
                return 0
            lax.fori_loop(0, NB // 2, _pair, 0)
            lb = NB - 1
            pltpu.make_async_copy(y_hbm.at[idx_t.at[lb]], rows0, sem0).wait()
            _scale_scatter(rows0, lb)
            plsc.subcore_barrier()

            for ch in range(5):
                node0 = t * NPT + ch * CH
                rout = (k * 4 + qq) * NNP + node0
                rt0 = (km2 * 4 + qq) * NNP + node0
                pltpu.sync_copy(y_s.at[pl.ds(node0, CH)], rows0)
                pltpu.sync_copy(y_hbm.at[pl.ds(rt0, CH)], rows1)

                def _comb(r, _):
                    for j in range(QF // 16):
                        sl = pl.ds(j * 16, 16)
                        rows0[r, sl] = alpha * rows0[r, sl] + gamma * rows1[r, sl]
                    return 0
                lax.fori_loop(0, CH, _comb, 0, unroll=4)
                pltpu.sync_copy(rows0, y_hbm.at[pl.ds(rout, CH)])
                pltpu.sync_copy(zb2, y_s.at[pl.ds(node0, CH)])

            _shift_idx(NNP if q == 0 else 3 * NNP)
            plsc.subcore_barrier()
        return 0

    lax.fori_loop(1, KDEG + 1, _kbody, 0)


def _sc_propagate(x4f, srcp, dstp, wp):
    mesh = plsc.VectorSubcoreMesh(core_axis_name="c", subcore_axis_name="s")
    kern = pl.kernel(
        _sc_body,
        out_type=jax.ShapeDtypeStruct((NROWS, QF), jnp.float32),
        mesh=mesh,
        compiler_params=pltpu.CompilerParams(use_tc_tiling_on_sc=False,
                                             needs_layout_passes=False),
        scratch_types=[
            pltpu.VMEM((NB, B), jnp.int32),
            pltpu.VMEM((NB, B), jnp.int32),
            pltpu.VMEM((NB, B), jnp.float32),
            pltpu.VMEM((NNP,), jnp.float32),
            pltpu.VMEM((B, QF), jnp.float32),
            pltpu.VMEM((B, QF), jnp.float32),
            pltpu.VMEM((B, QF), jnp.float32),
            pltpu.VMEM((NPT,), jnp.float32),
            pltpu.VMEM((B,), jnp.float32),
            pltpu.VMEM_SHARED((NNP, QF), jnp.float32),
            pltpu.VMEM_SHARED((NNP,), jnp.float32),
            pltpu.VMEM_SHARED((NNP,), jnp.float32),
            pltpu.SemaphoreType.DMA,
            pltpu.SemaphoreType.DMA,
        ],
    )
    return kern(x4f, srcp, dstp, wp)


def _head_body(y_ref, b_ref, coefs_ref, cw_ref, cb_ref, fcw_ref, fcb_ref,
               out_ref, acc_s, acc_c):
    i = pl.program_id(0)

    @pl.when(i == 0)
    def _():
        acc_s[...] = jnp.zeros_like(acc_s)
        acc_c[...] = jnp.zeros_like(acc_c)

    hi = jax.lax.Precision.HIGHEST
    y17 = y_ref[...]
    out16 = lax.dot_general(coefs_ref[...], y17.reshape(17, BN * 128),
                            (((1,), (0,)), ((), ())),
                            precision=hi).reshape(TT, BN, 128)
    h = None
    for dt in range(3):
        part = lax.dot_general(out16[dt:dt + 14].reshape(14 * BN, 128),
                               cw_ref[dt], (((1,), (1,)), ((), ())),
                               precision=hi)
        h = part if h is None else h + part
    h = h.reshape(14, BN, 64) + cb_ref[0][None, None, :]
    hp = h.reshape(7, 2, BN, 64)
    hp = jnp.maximum(hp[:, 0], hp[:, 1])
    hp = jnp.maximum(hp, 0.0)
    flat = hp.transpose(1, 0, 2).reshape(BN, 7 * 64)
    bat = b_ref[0, 0]
    onehot = (bat[None, :] == lax.broadcasted_iota(jnp.int32, (NGR, BN), 0))
    onehot = onehot.astype(jnp.float32)
    acc_s[...] += lax.dot_general(onehot, flat, (((1,), (0,)), ((), ())),
                                  precision=hi)
    acc_c[...] += jnp.sum(onehot, axis=1, keepdims=True).reshape(1, NGR)

    @pl.when(i == NBLK - 1)
    def _():
        cnt = jnp.maximum(acc_c[0], 1.0)
        gm = acc_s[...] / cnt[:, None]
        gm = jnp.maximum(gm, 0.0)
        logits = lax.dot_general(gm, fcw_ref[...], (((1,), (1,)), ((), ())),
                                 precision=hi) + fcb_ref[0][None, :]
        m = jnp.max(logits, axis=1, keepdims=True)
        e = logits - m
        out_ref[...] = e - jnp.log(jnp.sum(jnp.exp(e), axis=1, keepdims=True))


def _head(y17, batch3, coefs, cwt, cb2, fcw, fcb2):
    return pl.pallas_call(
        _head_body,
        grid=(NBLK,),
        in_specs=[
            pl.BlockSpec((17, BN, 128), lambda i: (0, i, 0)),
            pl.BlockSpec((1, 1, BN), lambda i: (i, 0, 0)),
            pl.BlockSpec((TT, KDEG + 1), lambda i: (0, 0)),
            pl.BlockSpec((3, 64, 128), lambda i: (0, 0, 0)),
            pl.BlockSpec((1, 64), lambda i: (0, 0)),
            pl.BlockSpec((10, 7 * 64), lambda i: (0, 0)),
            pl.BlockSpec((1, 10), lambda i: (0, 0)),
        ],
        out_specs=pl.BlockSpec((NGR, 10), lambda i: (0, 0)),
        out_shape=jax.ShapeDtypeStruct((NGR, 10), jnp.float32),
        scratch_shapes=[
            pltpu.VMEM((NGR, 7 * 64), jnp.float32),
            pltpu.VMEM((1, NGR), jnp.float32),
        ],
    )(y17, batch3, coefs, cwt, cb2, fcw, fcb2)


def kernel(x, edge_index, edge_weight, batch, coefs, conv_w, conv_b, fc_w, fc_b):
    src = edge_index[0].astype(jnp.int32)
    dst = edge_index[1].astype(jnp.int32)
    w = edge_weight.astype(jnp.float32)
    pad = EPTP - EPT
    srcp = jnp.pad(src.reshape(16, EPT), ((0, 0), (0, pad))).reshape(16, NB, B)
    dstp = jnp.pad(dst.reshape(16, EPT), ((0, 0), (0, pad))).reshape(16, NB, B)
    wp = jnp.pad(w.reshape(16, EPT), ((0, 0), (0, pad))).reshape(16 * NB * B)
    x4 = x.reshape(NN, 4, QF).transpose(1, 0, 2)
    x4f = jnp.pad(x4, ((0, 0), (0, NNP - NN), (0, 0))).reshape(4 * NNP, QF)

    yflat = _sc_propagate(x4f, srcp, dstp, wp)
    y17 = yflat.reshape(17, 4, NNP, QF).transpose(0, 2, 1, 3).reshape(17, NNP, 128)

    batchp = jnp.pad(batch.astype(jnp.int32), (0, NNP - NN),
                     constant_values=NGR)
    batch3 = batchp.reshape(NBLK, 1, BN)
    cwt = conv_w.transpose(2, 0, 1)
    cb2 = conv_b.reshape(1, 64)
    fcb2 = fc_b.reshape(1, 10)
    return _head(y17, batch3, coefs, cwt, cb2, fc_w, fcb2)


# speedup vs baseline: 5.7560x; 5.3833x over previous
import jax
import jax.numpy as jnp
from jax import lax
from jax.experimental import pallas as pl
from jax.experimental.pallas import tpu as pltpu
from jax.experimental.pallas import tpu_sc as plsc

NN = 10000
NNP = 10240
EE = 320000
QF = 32
EPT = 20000
B = 128
NB = EPT // B + (1 if EPT % B else 0)
EPTP = NB * B
NPT = NNP // 16
CH = 128
KDEG = 16
NROWS = 68 * NNP
TT = 16
NGR = 128
NBLK = 20
BN = NNP // NBLK


def _sc_body(x4f, srcp, dstp, wp, y_hbm,
             idx_t, dst_t, nrm_t, dis_v, rows0, rows1, zb2, zb1, wbuf,
             y_s, deg_s, dis_s, sem0, sem1):
    c = lax.axis_index("c")
    t = lax.axis_index("s")
    qbase = 2 * c

    pltpu.sync_copy(srcp.at[t], idx_t)
    pltpu.sync_copy(dstp.at[t], dst_t)

    zeros16 = jnp.zeros((16,), jnp.float32)

    def _zb2_body(r, _):
        for j in range(QF // 16):
            zb2[r, pl.ds(j * 16, 16)] = zeros16
        return 0
    lax.fori_loop(0, B, _zb2_body, 0, unroll=4)

    def _zb1_body(g, _):
        zb1[pl.ds(g * 16, 16)] = zeros16
        return 0
    lax.fori_loop(0, NPT // 16, _zb1_body, 0, unroll=4)

    pltpu.sync_copy(zb1, deg_s.at[pl.ds(t * NPT, NPT)])
    for ch in range(5):
        node0 = t * NPT + ch * CH
        pltpu.sync_copy(zb2, y_s.at[pl.ds(node0, CH)])
        for q in range(2):
            r0 = (qbase + q) * NNP + node0
            pltpu.sync_copy(x4f.at[pl.ds(r0, CH)], rows0)
            pltpu.sync_copy(rows0, y_hbm.at[pl.ds(r0, CH)])
    plsc.subcore_barrier()

    def _deg_body(b, _):
        pltpu.sync_copy(wp.at[pl.ds(t * EPTP + b * B, B)], wbuf)
        pltpu.sync_copy(wbuf, deg_s.at[dst_t.at[b]], add=True)
        return 0
    lax.fori_loop(0, NB, _deg_body, 0)
    plsc.subcore_barrier()

    pltpu.sync_copy(deg_s.at[pl.ds(t * NPT, NPT)], dis_v.at[pl.ds(0, NPT)])

    def _dis_body(g, _):
        v = dis_v[pl.ds(g * 16, 16)]
        i = lax.bitcast_convert_type(v, jnp.int32)
        i = 0x5F3759DF - lax.shift_right_arithmetic(i, 1)
        y = lax.bitcast_convert_type(i, jnp.float32)
        for _ in range(3):
            y = y * (1.5 - 0.5 * v * y * y)
        dis_v[pl.ds(g * 16, 16)] = jnp.where(v > 0.0, y, 0.0)
        return 0
    lax.fori_loop(0, NPT // 16, _dis_body, 0, unroll=4)
    pltpu.sync_copy(dis_v.at[pl.ds(0, NPT)], dis_s.at[pl.ds(t * NPT, NPT)])
    plsc.subcore_barrier()
    pltpu.sync_copy(dis_s, dis_v)

    def _nrm_body(b, _):
        pltpu.sync_copy(wp.at[pl.ds(t * EPTP + b * B, B)], wbuf)
        for g in range(8):
            sl = pl.ds(g * 16, 16)
            srcv = idx_t[b, sl]
            dstv = dst_t[b, sl]
            d1 = plsc.load_gather(dis_v, [srcv])
            d2 = plsc.load_gather(dis_v, [dstv])
            nrm_t[b, sl] = d1 * wbuf[sl] * d2
            idx_t[b, sl] = srcv + qbase * NNP
        return 0
    lax.fori_loop(0, NB, _nrm_body, 0)
    plsc.subcore_barrier()

    def _scale_scatter(buf, b):
        def _grp(g, _):
            nv = nrm_t[b, pl.ds(g * 16, 16)]
            for rr in range(16):
                row = g * 16 + rr
                s = nv[rr]
                for j in range(QF // 16):
                    sl = pl.ds(j * 16, 16)
                    buf[row, sl] = buf[row, sl] * s
            return 0
        lax.fori_loop(0, B // 16, _grp, 0)
        pltpu.sync_copy(buf, y_s.at[dst_t.at[b]], add=True)

    def _shift_idx(delta):
        def _shift(b, _):
            for g in range(8):
                sl = pl.ds(g * 16, 16)
                idx_t[b, sl] = idx_t[b, sl] + delta
            return 0
        lax.fori_loop(0, NB, _shift, 0)

    def _kbody(k, _):
        alpha = jnp.where(k == 1, -1.0, -2.0)
        gamma = jnp.where(k == 1, 0.0, -1.0)
        km2 = jnp.where(k == 1, 0, k - 2)
        for q in range(2):
            qq = qbase + q
            pltpu.async_copy(y_hbm.at[idx_t.at[0]], rows0, sem0)

            def _pair(p, _):
                b0 = 2 * p
                b1 = b0 + 1
                pltpu.async_copy(y_hbm.at[idx_t.at[b1]], rows1, sem1)
                pltpu.make_async_copy(y_hbm.at[idx_t.at[b0]], rows0, sem0).wait()
                _scale_scatter(rows0, b0)
                pltpu.async_copy(y_hbm.at[idx_t.at[b0 + 2]], rows0, sem0)
                pltpu.make_async_copy(y_hbm.at[idx_t.at[b1]], rows1, sem1).wait()
                _scale_scatter(rows1, b1)
                return 0
            lax.fori_loop(0, NB // 2, _pair, 0)
            lb = NB - 1
            pltpu.make_async_copy(y_hbm.at[idx_t.at[lb]], rows0, sem0).wait()
            _scale_scatter(rows0, lb)
            plsc.subcore_barrier()

            for ch in range(5):
                node0 = t * NPT + ch * CH
                rout = (k * 4 + qq) * NNP + node0
                rt0 = (km2 * 4 + qq) * NNP + node0
                pltpu.sync_copy(y_s.at[pl.ds(node0, CH)], rows0)
                pltpu.sync_copy(y_hbm.at[pl.ds(rt0, CH)], rows1)

                def _comb(r, _):
                    for j in range(QF // 16):
                        sl = pl.ds(j * 16, 16)
                        rows0[r, sl] = alpha * rows0[r, sl] + gamma * rows1[r, sl]
                    return 0
                lax.fori_loop(0, CH, _comb, 0, unroll=4)
                pltpu.sync_copy(rows0, y_hbm.at[pl.ds(rout, CH)])
                pltpu.sync_copy(zb2, y_s.at[pl.ds(node0, CH)])

            _shift_idx(NNP if q == 0 else 3 * NNP)
            plsc.subcore_barrier()
        return 0

    lax.fori_loop(1, KDEG + 1, _kbody, 0)


def _sc_propagate(x4f, srcp, dstp, wp):
    mesh = plsc.VectorSubcoreMesh(core_axis_name="c", subcore_axis_name="s")
    kern = pl.kernel(
        _sc_body,
        out_type=jax.ShapeDtypeStruct((NROWS, QF), jnp.float32),
        mesh=mesh,
        compiler_params=pltpu.CompilerParams(use_tc_tiling_on_sc=False,
                                             needs_layout_passes=False),
        scratch_types=[
            pltpu.VMEM((NB, B), jnp.int32),
            pltpu.VMEM((NB, B), jnp.int32),
            pltpu.VMEM((NB, B), jnp.float32),
            pltpu.VMEM((NNP,), jnp.float32),
            pltpu.VMEM((B, QF), jnp.float32),
            pltpu.VMEM((B, QF), jnp.float32),
            pltpu.VMEM((B, QF), jnp.float32),
            pltpu.VMEM((NPT,), jnp.float32),
            pltpu.VMEM((B,), jnp.float32),
            pltpu.VMEM_SHARED((NNP, QF), jnp.float32),
            pltpu.VMEM_SHARED((NNP,), jnp.float32),
            pltpu.VMEM_SHARED((NNP,), jnp.float32),
            pltpu.SemaphoreType.DMA,
            pltpu.SemaphoreType.DMA,
        ],
    )
    return kern(x4f, srcp, dstp, wp)


def _head_body(y_ref, b_ref, coefs_ref, cw_ref, cb_ref, fcw_ref, fcb_ref,
               out_ref, acc_s, acc_c):
    i = pl.program_id(0)

    @pl.when(i == 0)
    def _():
        acc_s[...] = jnp.zeros_like(acc_s)
        acc_c[...] = jnp.zeros_like(acc_c)

    hi = jax.lax.Precision.HIGHEST
    y17 = y_ref[...]
    out16 = lax.dot_general(coefs_ref[...], y17.reshape(17, BN * 128),
                            (((1,), (0,)), ((), ())),
                            precision=hi).reshape(TT, BN, 128)
    h = None
    for dt in range(3):
        part = lax.dot_general(out16[dt:dt + 14].reshape(14 * BN, 128),
                               cw_ref[dt], (((1,), (1,)), ((), ())),
                               precision=hi)
        h = part if h is None else h + part
    h = h.reshape(14, BN, 64) + cb_ref[0][None, None, :]
    hp = h.reshape(7, 2, BN, 64)
    hp = jnp.maximum(hp[:, 0], hp[:, 1])
    hp = jnp.maximum(hp, 0.0)
    flat = hp.transpose(1, 0, 2).reshape(BN, 7 * 64)
    bat = b_ref[0, 0]
    onehot = (bat[None, :] == lax.broadcasted_iota(jnp.int32, (NGR, BN), 0))
    onehot = onehot.astype(jnp.float32)
    acc_s[...] += lax.dot_general(onehot, flat, (((1,), (0,)), ((), ())),
                                  precision=hi)
    acc_c[...] += jnp.sum(onehot, axis=1, keepdims=True).reshape(1, NGR)

    @pl.when(i == NBLK - 1)
    def _():
        cnt = jnp.maximum(acc_c[0], 1.0)
        gm = acc_s[...] / cnt[:, None]
        gm = jnp.maximum(gm, 0.0)
        logits = lax.dot_general(gm, fcw_ref[...], (((1,), (1,)), ((), ())),
                                 precision=hi) + fcb_ref[0][None, :]
        m = jnp.max(logits, axis=1, keepdims=True)
        e = logits - m
        out_ref[...] = e - jnp.log(jnp.sum(jnp.exp(e), axis=1, keepdims=True))


def _head(y17, batch3, coefs, cwt, cb2, fcw, fcb2):
    return pl.pallas_call(
        _head_body,
        grid=(NBLK,),
        in_specs=[
            pl.BlockSpec((17, BN, 128), lambda i: (0, i, 0)),
            pl.BlockSpec((1, 1, BN), lambda i: (i, 0, 0)),
            pl.BlockSpec((TT, KDEG + 1), lambda i: (0, 0)),
            pl.BlockSpec((3, 64, 128), lambda i: (0, 0, 0)),
            pl.BlockSpec((1, 64), lambda i: (0, 0)),
            pl.BlockSpec((10, 7 * 64), lambda i: (0, 0)),
            pl.BlockSpec((1, 10), lambda i: (0, 0)),
        ],
        out_specs=pl.BlockSpec((NGR, 10), lambda i: (0, 0)),
        out_shape=jax.ShapeDtypeStruct((NGR, 10), jnp.float32),
        scratch_shapes=[
            pltpu.VMEM((NGR, 7 * 64), jnp.float32),
            pltpu.VMEM((1, NGR), jnp.float32),
        ],
    )(y17, batch3, coefs, cwt, cb2, fcw, fcb2)


def kernel(x, edge_index, edge_weight, batch, coefs, conv_w, conv_b, fc_w, fc_b):
    src = edge_index[0].astype(jnp.int32)
    dst = edge_index[1].astype(jnp.int32)
    w = edge_weight.astype(jnp.float32)
    pad = EPTP - EPT
    srcp = jnp.pad(src.reshape(16, EPT), ((0, 0), (0, pad))).reshape(16, NB, B)
    dstp = jnp.pad(dst.reshape(16, EPT), ((0, 0), (0, pad))).reshape(16, NB, B)
    wp = jnp.pad(w.reshape(16, EPT), ((0, 0), (0, pad))).reshape(16 * NB * B)
    x4 = x.reshape(NN, 4, QF).transpose(1, 0, 2)
    x4f = jnp.pad(x4, ((0, 0), (0, NNP - NN), (0, 0))).reshape(4 * NNP, QF)

    yflat = _sc_propagate(x4f, srcp, dstp, wp)
    y17 = yflat.reshape(17, 4, NNP, QF).transpose(0, 2, 1, 3).reshape(17, NNP, 128)

    batchp = jnp.pad(batch.astype(jnp.int32), (0, NNP - NN),
                     constant_values=NGR)
    batch3 = batchp.reshape(NBLK, 1, BN)
    cwt = conv_w.transpose(2, 0, 1)
    cb2 = conv_b.reshape(1, 64)
    fcb2 = fc_b.reshape(1, 10)
    return _head(y17, batch3, coefs, cwt, cb2, fc_w, fcb2)


# speedup vs baseline: 6.0751x; 1.0554x over previous
import jax
import jax.numpy as jnp
from jax import lax
from jax.experimental import pallas as pl
from jax.experimental.pallas import tpu as pltpu
from jax.experimental.pallas import tpu_sc as plsc

NN = 10000
NNP = 10240
EE = 320000
QF = 64
EPT = 20000
B = 64
NB = EPT // B + (1 if EPT % B else 0)
EPTP = NB * B
NPT = NNP // 16
CH = 64
NCHK = NPT // CH
KDEG = 16
NROWS = 34 * NNP
TT = 16
NGR = 128
NBLK = 20
BN = NNP // NBLK


def _sc_body(x2f, srcp, dstp, wp, y_hbm,
             idx_t, dst_t, nrm_t, dis_v, rows0, rows1, zb2, zb1, wbuf,
             y_s, deg_s, dis_s, sem0, sem1):
    c = lax.axis_index("c")
    t = lax.axis_index("s")

    pltpu.sync_copy(srcp.at[t], idx_t)
    pltpu.sync_copy(dstp.at[t], dst_t)

    zeros16 = jnp.zeros((16,), jnp.float32)

    def _zb2_body(r, _):
        for j in range(QF // 16):
            zb2[r, pl.ds(j * 16, 16)] = zeros16
        return 0
    lax.fori_loop(0, B, _zb2_body, 0, unroll=4)

    def _zb1_body(g, _):
        zb1[pl.ds(g * 16, 16)] = zeros16
        return 0
    lax.fori_loop(0, NPT // 16, _zb1_body, 0, unroll=4)

    pltpu.sync_copy(zb1, deg_s.at[pl.ds(t * NPT, NPT)])
    for ch in range(NCHK):
        node0 = t * NPT + ch * CH
        pltpu.sync_copy(zb2, y_s.at[pl.ds(node0, CH)])
        r0 = c * NNP + node0
        pltpu.sync_copy(x2f.at[pl.ds(r0, CH)], rows0)
        pltpu.sync_copy(rows0, y_hbm.at[pl.ds(r0, CH)])
    plsc.subcore_barrier()

    def _deg_body(b, _):
        pltpu.sync_copy(wp.at[pl.ds(t * EPTP + b * B, B)], wbuf)
        pltpu.sync_copy(wbuf, deg_s.at[dst_t.at[b]], add=True)
        return 0
    lax.fori_loop(0, NB, _deg_body, 0)
    plsc.subcore_barrier()

    pltpu.sync_copy(deg_s.at[pl.ds(t * NPT, NPT)], dis_v.at[pl.ds(0, NPT)])

    def _dis_body(g, _):
        v = dis_v[pl.ds(g * 16, 16)]
        i = lax.bitcast_convert_type(v, jnp.int32)
        i = 0x5F3759DF - lax.shift_right_arithmetic(i, 1)
        y = lax.bitcast_convert_type(i, jnp.float32)
        for _ in range(3):
            y = y * (1.5 - 0.5 * v * y * y)
        dis_v[pl.ds(g * 16, 16)] = jnp.where(v > 0.0, y, 0.0)
        return 0
    lax.fori_loop(0, NPT // 16, _dis_body, 0, unroll=4)
    pltpu.sync_copy(dis_v.at[pl.ds(0, NPT)], dis_s.at[pl.ds(t * NPT, NPT)])
    plsc.subcore_barrier()
    pltpu.sync_copy(dis_s, dis_v)

    def _nrm_body(b, _):
        pltpu.sync_copy(wp.at[pl.ds(t * EPTP + b * B, B)], wbuf)
        for g in range(B // 16):
            sl = pl.ds(g * 16, 16)
            srcv = idx_t[b, sl]
            dstv = dst_t[b, sl]
            d1 = plsc.load_gather(dis_v, [srcv])
            d2 = plsc.load_gather(dis_v, [dstv])
            nrm_t[b, sl] = d1 * wbuf[sl] * d2
            idx_t[b, sl] = srcv + c * NNP
        return 0
    lax.fori_loop(0, NB, _nrm_body, 0)
    plsc.subcore_barrier()

    def _scale_scatter(buf, b):
        def _grp(g, _):
            nv = nrm_t[b, pl.ds(g * 16, 16)]
            for rr in range(16):
                row = g * 16 + rr
                s = nv[rr]
                for j in range(QF // 16):
                    sl = pl.ds(j * 16, 16)
                    buf[row, sl] = buf[row, sl] * s
            return 0
        lax.fori_loop(0, B // 16, _grp, 0)
        pltpu.sync_copy(buf, y_s.at[dst_t.at[b]], add=True)

    def _shift_idx(delta):
        def _shift(b, _):
            for g in range(B // 16):
                sl = pl.ds(g * 16, 16)
                idx_t[b, sl] = idx_t[b, sl] + delta
            return 0
        lax.fori_loop(0, NB, _shift, 0)

    def _kbody(k, _):
        alpha = jnp.where(k == 1, -1.0, -2.0)
        gamma = jnp.where(k == 1, 0.0, -1.0)
        km2 = jnp.where(k == 1, 0, k - 2)
        pltpu.async_copy(y_hbm.at[idx_t.at[0]], rows0, sem0)

        def _pair(p, _):
            b0 = 2 * p
            b1 = b0 + 1
            pltpu.async_copy(y_hbm.at[idx_t.at[b1]], rows1, sem1)
            pltpu.make_async_copy(y_hbm.at[idx_t.at[b0]], rows0, sem0).wait()
            _scale_scatter(rows0, b0)
            pltpu.async_copy(y_hbm.at[idx_t.at[b0 + 2]], rows0, sem0)
            pltpu.make_async_copy(y_hbm.at[idx_t.at[b1]], rows1, sem1).wait()
            _scale_scatter(rows1, b1)
            return 0
        lax.fori_loop(0, NB // 2, _pair, 0)
        lb = NB - 1
        pltpu.make_async_copy(y_hbm.at[idx_t.at[lb]], rows0, sem0).wait()
        _scale_scatter(rows0, lb)
        plsc.subcore_barrier()

        for ch in range(NCHK):
            node0 = t * NPT + ch * CH
            rout = (k * 2 + c) * NNP + node0
            rt0 = (km2 * 2 + c) * NNP + node0
            pltpu.sync_copy(y_s.at[pl.ds(node0, CH)], rows0)
            pltpu.sync_copy(y_hbm.at[pl.ds(rt0, CH)], rows1)

            def _comb(r, _):
                for j in range(QF // 16):
                    sl = pl.ds(j * 16, 16)
                    rows0[r, sl] = alpha * rows0[r, sl] + gamma * rows1[r, sl]
                return 0
            lax.fori_loop(0, CH, _comb, 0, unroll=4)
            pltpu.sync_copy(rows0, y_hbm.at[pl.ds(rout, CH)])
            pltpu.sync_copy(zb2, y_s.at[pl.ds(node0, CH)])

        _shift_idx(2 * NNP)
        plsc.subcore_barrier()
        return 0

    lax.fori_loop(1, KDEG + 1, _kbody, 0)


def _sc_propagate(x2f, srcp, dstp, wp):
    mesh = plsc.VectorSubcoreMesh(core_axis_name="c", subcore_axis_name="s")
    kern = pl.kernel(
        _sc_body,
        out_type=jax.ShapeDtypeStruct((NROWS, QF), jnp.float32),
        mesh=mesh,
        compiler_params=pltpu.CompilerParams(use_tc_tiling_on_sc=False,
                                             needs_layout_passes=False),
        scratch_types=[
            pltpu.VMEM((NB, B), jnp.int32),
            pltpu.VMEM((NB, B), jnp.int32),
            pltpu.VMEM((NB, B), jnp.float32),
            pltpu.VMEM((NNP,), jnp.float32),
            pltpu.VMEM((B, QF), jnp.float32),
            pltpu.VMEM((B, QF), jnp.float32),
            pltpu.VMEM((B, QF), jnp.float32),
            pltpu.VMEM((NPT,), jnp.float32),
            pltpu.VMEM((B,), jnp.float32),
            pltpu.VMEM_SHARED((NNP, QF), jnp.float32),
            pltpu.VMEM_SHARED((NNP,), jnp.float32),
            pltpu.VMEM_SHARED((NNP,), jnp.float32),
            pltpu.SemaphoreType.DMA,
            pltpu.SemaphoreType.DMA,
        ],
    )
    return kern(x2f, srcp, dstp, wp)


def _head_body(y_ref, b_ref, coefs_ref, cw_ref, cb_ref, fcw_ref, fcb_ref,
               out_ref, acc_s, acc_c):
    i = pl.program_id(0)

    @pl.when(i == 0)
    def _():
        acc_s[...] = jnp.zeros_like(acc_s)
        acc_c[...] = jnp.zeros_like(acc_c)

    hi = jax.lax.Precision.HIGHEST
    y17 = y_ref[...]
    out16 = lax.dot_general(coefs_ref[...], y17.reshape(17, BN * 128),
                            (((1,), (0,)), ((), ())),
                            precision=hi).reshape(TT, BN, 128)
    h = None
    for dt in range(3):
        part = lax.dot_general(out16[dt:dt + 14].reshape(14 * BN, 128),
                               cw_ref[dt], (((1,), (1,)), ((), ())),
                               precision=hi)
        h = part if h is None else h + part
    h = h.reshape(14, BN, 64) + cb_ref[0][None, None, :]
    hp = h.reshape(7, 2, BN, 64)
    hp = jnp.maximum(hp[:, 0], hp[:, 1])
    hp = jnp.maximum(hp, 0.0)
    flat = hp.transpose(1, 0, 2).reshape(BN, 7 * 64)
    bat = b_ref[0, 0]
    onehot = (bat[None, :] == lax.broadcasted_iota(jnp.int32, (NGR, BN), 0))
    onehot = onehot.astype(jnp.float32)
    acc_s[...] += lax.dot_general(onehot, flat, (((1,), (0,)), ((), ())),
                                  precision=hi)
    acc_c[...] += jnp.sum(onehot, axis=1, keepdims=True).reshape(1, NGR)

    @pl.when(i == NBLK - 1)
    def _():
        cnt = jnp.maximum(acc_c[0], 1.0)
        gm = acc_s[...] / cnt[:, None]
        gm = jnp.maximum(gm, 0.0)
        logits = lax.dot_general(gm, fcw_ref[...], (((1,), (1,)), ((), ())),
                                 precision=hi) + fcb_ref[0][None, :]
        m = jnp.max(logits, axis=1, keepdims=True)
        e = logits - m
        out_ref[...] = e - jnp.log(jnp.sum(jnp.exp(e), axis=1, keepdims=True))


def _head(y17, batch3, coefs, cwt, cb2, fcw, fcb2):
    return pl.pallas_call(
        _head_body,
        grid=(NBLK,),
        in_specs=[
            pl.BlockSpec((17, BN, 128), lambda i: (0, i, 0)),
            pl.BlockSpec((1, 1, BN), lambda i: (i, 0, 0)),
            pl.BlockSpec((TT, KDEG + 1), lambda i: (0, 0)),
            pl.BlockSpec((3, 64, 128), lambda i: (0, 0, 0)),
            pl.BlockSpec((1, 64), lambda i: (0, 0)),
            pl.BlockSpec((10, 7 * 64), lambda i: (0, 0)),
            pl.BlockSpec((1, 10), lambda i: (0, 0)),
        ],
        out_specs=pl.BlockSpec((NGR, 10), lambda i: (0, 0)),
        out_shape=jax.ShapeDtypeStruct((NGR, 10), jnp.float32),
        scratch_shapes=[
            pltpu.VMEM((NGR, 7 * 64), jnp.float32),
            pltpu.VMEM((1, NGR), jnp.float32),
        ],
    )(y17, batch3, coefs, cwt, cb2, fcw, fcb2)


def kernel(x, edge_index, edge_weight, batch, coefs, conv_w, conv_b, fc_w, fc_b):
    src = edge_index[0].astype(jnp.int32)
    dst = edge_index[1].astype(jnp.int32)
    w = edge_weight.astype(jnp.float32)
    pad = EPTP - EPT
    srcp = jnp.pad(src.reshape(16, EPT), ((0, 0), (0, pad))).reshape(16, NB, B)
    dstp = jnp.pad(dst.reshape(16, EPT), ((0, 0), (0, pad))).reshape(16, NB, B)
    wp = jnp.pad(w.reshape(16, EPT), ((0, 0), (0, pad))).reshape(16 * NB * B)
    x2 = x.reshape(NN, 2, QF).transpose(1, 0, 2)
    x2f = jnp.pad(x2, ((0, 0), (0, NNP - NN), (0, 0))).reshape(2 * NNP, QF)

    yflat = _sc_propagate(x2f, srcp, dstp, wp)
    y17 = yflat.reshape(17, 2, NNP, QF).transpose(0, 2, 1, 3).reshape(17, NNP, 128)

    batchp = jnp.pad(batch.astype(jnp.int32), (0, NNP - NN),
                     constant_values=NGR)
    batch3 = batchp.reshape(NBLK, 1, BN)
    cwt = conv_w.transpose(2, 0, 1)
    cb2 = conv_b.reshape(1, 64)
    fcb2 = fc_b.reshape(1, 10)
    return _head(y17, batch3, coefs, cwt, cb2, fc_w, fcb2)


# speedup vs baseline: 6.5250x; 1.0741x over previous
import jax
import jax.numpy as jnp
from jax import lax
from jax.experimental import pallas as pl
from jax.experimental.pallas import tpu as pltpu
from jax.experimental.pallas import tpu_sc as plsc

NN = 10000
NNP = 10240
EE = 320000
QF = 64
EPT = 20000
B = 64
NB = EPT // B + (1 if EPT % B else 0)
EPTP = NB * B
NPT = NNP // 16
CH = 64
NCHK = NPT // CH
KDEG = 16
NROWS = 34 * NNP
TT = 16
NGR = 128
NBLK = 20
BN = NNP // NBLK


def _sc_body(x2f, srcp, dstp, wp, y_hbm,
             idx_t, dst_t, nrm_t, dis_v, rows0, rows1, sbuf, zb2, zb1, wbuf,
             y_s, deg_s, dis_s, sem0, sem1):
    c = lax.axis_index("c")
    t = lax.axis_index("s")

    pltpu.sync_copy(srcp.at[t], idx_t)
    pltpu.sync_copy(dstp.at[t], dst_t)

    zeros16 = jnp.zeros((16,), jnp.float32)

    def _zb2_body(r, _):
        for j in range(QF // 16):
            zb2[r, pl.ds(j * 16, 16)] = zeros16
        return 0
    lax.fori_loop(0, B, _zb2_body, 0, unroll=4)

    def _zb1_body(g, _):
        zb1[pl.ds(g * 16, 16)] = zeros16
        return 0
    lax.fori_loop(0, NPT // 16, _zb1_body, 0, unroll=4)

    pltpu.sync_copy(zb1, deg_s.at[pl.ds(t * NPT, NPT)])
    for ch in range(NCHK):
        node0 = t * NPT + ch * CH
        pltpu.sync_copy(zb2, y_s.at[pl.ds(node0, CH)])
        r0 = c * NNP + node0
        pltpu.sync_copy(x2f.at[pl.ds(r0, CH)], rows0)
        pltpu.sync_copy(rows0, y_hbm.at[pl.ds(r0, CH)])
    plsc.subcore_barrier()

    def _deg_body(b, _):
        pltpu.sync_copy(wp.at[pl.ds(t * EPTP + b * B, B)], wbuf)
        pltpu.sync_copy(wbuf, deg_s.at[dst_t.at[b]], add=True)
        return 0
    lax.fori_loop(0, NB, _deg_body, 0)
    plsc.subcore_barrier()

    pltpu.sync_copy(deg_s.at[pl.ds(t * NPT, NPT)], dis_v.at[pl.ds(0, NPT)])

    def _dis_body(g, _):
        v = dis_v[pl.ds(g * 16, 16)]
        i = lax.bitcast_convert_type(v, jnp.int32)
        i = 0x5F3759DF - lax.shift_right_arithmetic(i, 1)
        y = lax.bitcast_convert_type(i, jnp.float32)
        for _ in range(3):
            y = y * (1.5 - 0.5 * v * y * y)
        dis_v[pl.ds(g * 16, 16)] = jnp.where(v > 0.0, y, 0.0)
        return 0
    lax.fori_loop(0, NPT // 16, _dis_body, 0, unroll=4)
    pltpu.sync_copy(dis_v.at[pl.ds(0, NPT)], dis_s.at[pl.ds(t * NPT, NPT)])
    plsc.subcore_barrier()
    pltpu.sync_copy(dis_s, dis_v)

    def _nrm_body(b, _):
        pltpu.sync_copy(wp.at[pl.ds(t * EPTP + b * B, B)], wbuf)
        for g in range(B // 16):
            sl = pl.ds(g * 16, 16)
            srcv = idx_t[b, sl]
            dstv = dst_t[b, sl]
            d1 = plsc.load_gather(dis_v, [srcv])
            d2 = plsc.load_gather(dis_v, [dstv])
            nrm_t[b, sl] = d1 * wbuf[sl] * d2
            idx_t[b, sl] = srcv + c * NNP
        return 0
    lax.fori_loop(0, NB, _nrm_body, 0)
    plsc.subcore_barrier()

    def _scale_scatter(buf, b):
        def _grp(g, _):
            nv = nrm_t[b, pl.ds(g * 16, 16)]
            for rr in range(16):
                row = g * 16 + rr
                s = nv[rr]
                for j in range(QF // 16):
                    sl = pl.ds(j * 16, 16)
                    sbuf[row, sl] = buf[row, sl] * s
            return 0
        lax.fori_loop(0, B // 16, _grp, 0)
        pltpu.sync_copy(sbuf, y_s.at[dst_t.at[b]], add=True)

    def _shift_idx(delta):
        def _shift(b, _):
            for g in range(B // 16):
                sl = pl.ds(g * 16, 16)
                idx_t[b, sl] = idx_t[b, sl] + delta
            return 0
        lax.fori_loop(0, NB, _shift, 0)

    def _kbody(k, _):
        alpha = jnp.where(k == 1, -1.0, -2.0)
        gamma = jnp.where(k == 1, 0.0, -1.0)
        km2 = jnp.where(k == 1, 0, k - 2)
        pltpu.async_copy(y_hbm.at[idx_t.at[0]], rows0, sem0)

        def _pair(p, _):
            b0 = 2 * p
            b1 = b0 + 1
            pltpu.async_copy(y_hbm.at[idx_t.at[b1]], rows1, sem1)
            pltpu.make_async_copy(y_hbm.at[idx_t.at[b0]], rows0, sem0).wait()
            _scale_scatter(rows0, b0)
            pltpu.async_copy(y_hbm.at[idx_t.at[b0 + 2]], rows0, sem0)
            pltpu.make_async_copy(y_hbm.at[idx_t.at[b1]], rows1, sem1).wait()
            _scale_scatter(rows1, b1)
            return 0
        lax.fori_loop(0, NB // 2, _pair, 0)
        lb = NB - 1
        pltpu.make_async_copy(y_hbm.at[idx_t.at[lb]], rows0, sem0).wait()
        _scale_scatter(rows0, lb)
        plsc.subcore_barrier()

        for ch in range(NCHK):
            node0 = t * NPT + ch * CH
            rout = (k * 2 + c) * NNP + node0
            rt0 = (km2 * 2 + c) * NNP + node0
            pltpu.sync_copy(y_s.at[pl.ds(node0, CH)], rows0)
            pltpu.sync_copy(y_hbm.at[pl.ds(rt0, CH)], rows1)

            def _comb(r, _):
                for j in range(QF // 16):
                    sl = pl.ds(j * 16, 16)
                    rows0[r, sl] = alpha * rows0[r, sl] + gamma * rows1[r, sl]
                return 0
            lax.fori_loop(0, CH, _comb, 0, unroll=4)
            pltpu.sync_copy(rows0, y_hbm.at[pl.ds(rout, CH)])
            pltpu.sync_copy(zb2, y_s.at[pl.ds(node0, CH)])

        _shift_idx(2 * NNP)
        plsc.subcore_barrier()
        return 0

    lax.fori_loop(1, KDEG + 1, _kbody, 0)


def _sc_propagate(x2f, srcp, dstp, wp):
    mesh = plsc.VectorSubcoreMesh(core_axis_name="c", subcore_axis_name="s")
    kern = pl.kernel(
        _sc_body,
        out_type=jax.ShapeDtypeStruct((NROWS, QF), jnp.float32),
        mesh=mesh,
        compiler_params=pltpu.CompilerParams(use_tc_tiling_on_sc=False,
                                             needs_layout_passes=False),
        scratch_types=[
            pltpu.VMEM((NB, B), jnp.int32),
            pltpu.VMEM((NB, B), jnp.int32),
            pltpu.VMEM((NB, B), jnp.float32),
            pltpu.VMEM((NNP,), jnp.float32),
            pltpu.VMEM((B, QF), jnp.float32),
            pltpu.VMEM((B, QF), jnp.float32),
            pltpu.VMEM((B, QF), jnp.float32),
            pltpu.VMEM((B, QF), jnp.float32),
            pltpu.VMEM((NPT,), jnp.float32),
            pltpu.VMEM((B,), jnp.float32),
            pltpu.VMEM_SHARED((NNP, QF), jnp.float32),
            pltpu.VMEM_SHARED((NNP,), jnp.float32),
            pltpu.VMEM_SHARED((NNP,), jnp.float32),
            pltpu.SemaphoreType.DMA,
            pltpu.SemaphoreType.DMA,
        ],
    )
    return kern(x2f, srcp, dstp, wp)


def _head_body(y_ref, b_ref, coefs_ref, cw_ref, cb_ref, fcw_ref, fcb_ref,
               out_ref, acc_s, acc_c):
    i = pl.program_id(0)

    @pl.when(i == 0)
    def _():
        acc_s[...] = jnp.zeros_like(acc_s)
        acc_c[...] = jnp.zeros_like(acc_c)

    hi = jax.lax.Precision.HIGHEST
    y17 = y_ref[...]
    out16 = lax.dot_general(coefs_ref[...], y17.reshape(17, BN * 128),
                            (((1,), (0,)), ((), ())),
                            precision=hi).reshape(TT, BN, 128)
    h = None
    for dt in range(3):
        part = lax.dot_general(out16[dt:dt + 14].reshape(14 * BN, 128),
                               cw_ref[dt], (((1,), (1,)), ((), ())),
                               precision=hi)
        h = part if h is None else h + part
    h = h.reshape(14, BN, 64) + cb_ref[0][None, None, :]
    hp = h.reshape(7, 2, BN, 64)
    hp = jnp.maximum(hp[:, 0], hp[:, 1])
    hp = jnp.maximum(hp, 0.0)
    flat = hp.transpose(1, 0, 2).reshape(BN, 7 * 64)
    bat = b_ref[0, 0]
    onehot = (bat[None, :] == lax.broadcasted_iota(jnp.int32, (NGR, BN), 0))
    onehot = onehot.astype(jnp.float32)
    acc_s[...] += lax.dot_general(onehot, flat, (((1,), (0,)), ((), ())),
                                  precision=hi)
    acc_c[...] += jnp.sum(onehot, axis=1, keepdims=True).reshape(1, NGR)

    @pl.when(i == NBLK - 1)
    def _():
        cnt = jnp.maximum(acc_c[0], 1.0)
        gm = acc_s[...] / cnt[:, None]
        gm = jnp.maximum(gm, 0.0)
        logits = lax.dot_general(gm, fcw_ref[...], (((1,), (1,)), ((), ())),
                                 precision=hi) + fcb_ref[0][None, :]
        m = jnp.max(logits, axis=1, keepdims=True)
        e = logits - m
        out_ref[...] = e - jnp.log(jnp.sum(jnp.exp(e), axis=1, keepdims=True))


def _head(y17, batch3, coefs, cwt, cb2, fcw, fcb2):
    return pl.pallas_call(
        _head_body,
        grid=(NBLK,),
        in_specs=[
            pl.BlockSpec((17, BN, 128), lambda i: (0, i, 0)),
            pl.BlockSpec((1, 1, BN), lambda i: (i, 0, 0)),
            pl.BlockSpec((TT, KDEG + 1), lambda i: (0, 0)),
            pl.BlockSpec((3, 64, 128), lambda i: (0, 0, 0)),
            pl.BlockSpec((1, 64), lambda i: (0, 0)),
            pl.BlockSpec((10, 7 * 64), lambda i: (0, 0)),
            pl.BlockSpec((1, 10), lambda i: (0, 0)),
        ],
        out_specs=pl.BlockSpec((NGR, 10), lambda i: (0, 0)),
        out_shape=jax.ShapeDtypeStruct((NGR, 10), jnp.float32),
        scratch_shapes=[
            pltpu.VMEM((NGR, 7 * 64), jnp.float32),
            pltpu.VMEM((1, NGR), jnp.float32),
        ],
    )(y17, batch3, coefs, cwt, cb2, fcw, fcb2)


def kernel(x, edge_index, edge_weight, batch, coefs, conv_w, conv_b, fc_w, fc_b):
    src = edge_index[0].astype(jnp.int32)
    dst = edge_index[1].astype(jnp.int32)
    w = edge_weight.astype(jnp.float32)
    pad = EPTP - EPT
    srcp = jnp.pad(src.reshape(16, EPT), ((0, 0), (0, pad))).reshape(16, NB, B)
    dstp = jnp.pad(dst.reshape(16, EPT), ((0, 0), (0, pad))).reshape(16, NB, B)
    wp = jnp.pad(w.reshape(16, EPT), ((0, 0), (0, pad))).reshape(16 * NB * B)
    x2 = x.reshape(NN, 2, QF).transpose(1, 0, 2)
    x2f = jnp.pad(x2, ((0, 0), (0, NNP - NN), (0, 0))).reshape(2 * NNP, QF)

    yflat = _sc_propagate(x2f, srcp, dstp, wp)
    y17 = yflat.reshape(17, 2, NNP, QF).transpose(0, 2, 1, 3).reshape(17, NNP, 128)

    batchp = jnp.pad(batch.astype(jnp.int32), (0, NNP - NN),
                     constant_values=NGR)
    batch3 = batchp.reshape(NBLK, 1, BN)
    cwt = conv_w.transpose(2, 0, 1)
    cb2 = conv_b.reshape(1, 64)
    fcb2 = fc_b.reshape(1, 10)
    return _head(y17, batch3, coefs, cwt, cb2, fc_w, fcb2)
